# Initial kernel scaffold; baseline (speedup 1.0000x reference)
import jax
import jax.numpy as jnp
from jax.experimental import pallas as pl


def kernel(relative_bias_weight, query_length, key_length):
    raise NotImplementedError("write your pallas kernel here")


# speedup vs baseline: 169.0913x; 169.0913x over previous
import math

import jax
import jax.numpy as jnp
from jax.experimental import pallas as pl
from jax.experimental.pallas import tpu as pltpu

N_BUCKETS = 32
MAX_DISTANCE = 128
N_HEAD = 16
Q_LEN = 2048
K_LEN = 2048

BQ = 512
D8_W = 4224
D128_W = 4096


def _bias_block_kernel(delta_ref, wt_ref, out_ref, d8_ref, d128_ref):
    qb = pl.program_id(1)
    delta = delta_ref[0]

    @pl.when(qb == 0)
    def _build_tables():
        i = jax.lax.broadcasted_iota(jnp.int32, (8, D8_W), 0)
        u = jax.lax.broadcasted_iota(jnp.int32, (8, D8_W), 1)
        rel = (u - i) - (Q_LEN - 1) + delta

        half = N_BUCKETS // 2
        bucket = jnp.where(rel > 0, half, 0).astype(jnp.int32)
        arel = jnp.abs(rel)
        max_exact = half // 2
        is_small = arel < max_exact
        me = max_exact - 1
        nb = half - 1
        safe = jnp.maximum(arel.astype(jnp.float32), 1.0)
        rp_large = me + (
            jnp.log(safe / me) / math.log(MAX_DISTANCE / me) * (nb - me)
        ).astype(jnp.int32)
        rp_large = jnp.minimum(rp_large, nb)
        bucket = bucket + jnp.where(is_small, arel, rp_large)

        w_row = wt_ref[0, 0, :]
        vals = jnp.zeros((8, D8_W), jnp.float32)
        for b in range(N_BUCKETS):
            vals = jnp.where(bucket == b, w_row[b], vals)
        d8_ref[:, :] = vals

        for m in range(16):
            d128_ref[pl.ds(8 * m, 8), :] = d8_ref[:, pl.ds(127 - 8 * m, D128_W)]

    q0 = qb * BQ
    for p in range(BQ // 128):
        s = (Q_LEN - 128) - q0 - 128 * p
        out_ref[0, 0, pl.ds(128 * p, 128), :] = d128_ref[:, pl.ds(s, K_LEN)]


def kernel(relative_bias_weight, query_length, key_length):
    delta = (
        jnp.asarray(key_length, jnp.int32) - K_LEN
        - (jnp.asarray(query_length, jnp.int32) - Q_LEN)
    ).reshape(1)
    wt = relative_bias_weight.T.reshape(N_HEAD, 1, N_BUCKETS)

    out = pl.pallas_call(
        _bias_block_kernel,
        grid=(N_HEAD, Q_LEN // BQ),
        in_specs=[
            pl.BlockSpec(memory_space=pltpu.SMEM),
            pl.BlockSpec((1, 1, N_BUCKETS), lambda h, qb: (h, 0, 0)),
        ],
        out_specs=pl.BlockSpec((1, 1, BQ, K_LEN), lambda h, qb: (0, h, qb, 0)),
        out_shape=jax.ShapeDtypeStruct((1, N_HEAD, Q_LEN, K_LEN), jnp.float32),
        scratch_shapes=[
            pltpu.VMEM((8, D8_W), jnp.float32),
            pltpu.VMEM((128, D128_W), jnp.float32),
        ],
        compiler_params=pltpu.CompilerParams(
            dimension_semantics=("parallel", "arbitrary"),
        ),
    )(delta, wt)
    return out
